# Initial kernel scaffold; baseline (speedup 1.0000x reference)
import jax
import jax.numpy as jnp
from jax.experimental import pallas as pl


def kernel(rpn_class, rpn_bbox, anchors):
    raise NotImplementedError("write your pallas kernel here")


# speedup vs baseline: 10.5458x; 10.5458x over previous
import jax
import jax.numpy as jnp
import numpy as np
from jax.experimental import pallas as pl

_PRE = 6000
_NPROP = 1000
_THR = 0.7
_PAD = 6144
_ROWS = 48
_BIG = 1 << 30


def _nms_kernel(a_ref, d_ref, y1_ref, x1_ref, y2_ref, x2_ref):
    row = jax.lax.broadcasted_iota(jnp.int32, (_ROWS, 128), 0)
    col = jax.lax.broadcasted_iota(jnp.int32, (_ROWS, 128), 1)
    idx = row * 128 + col
    row_o = jax.lax.broadcasted_iota(jnp.int32, (8, 128), 0)
    col_o = jax.lax.broadcasted_iota(jnp.int32, (8, 128), 1)
    idx_o = row_o * 128 + col_o

    boxes = []
    for b in range(4):
        ay1, ax1, ay2, ax2 = (a_ref[b, c] for c in range(4))
        dy = d_ref[b, 0] * 0.1
        dx = d_ref[b, 1] * 0.1
        dh = d_ref[b, 2] * 0.2
        dw = d_ref[b, 3] * 0.2
        h = ay2 - ay1
        w = ax2 - ax1
        cy = ay1 + 0.5 * h + dy * h
        cx = ax1 + 0.5 * w + dx * w
        h = h * jnp.exp(dh)
        w = w * jnp.exp(dw)
        y1 = cy - 0.5 * h
        x1 = cx - 0.5 * w
        y2 = y1 + h
        x2 = x1 + w
        y1 = jnp.clip(y1, 0.0, 1.0)
        x1 = jnp.clip(x1, 0.0, 1.0)
        y2 = jnp.clip(y2, 0.0, 1.0)
        x2 = jnp.clip(x2, 0.0, 1.0)
        area = jnp.maximum(y2 - y1, 0.0) * jnp.maximum(x2 - x1, 0.0)
        boxes.append((y1, x1, y2, x2, area))

    for ref in (y1_ref, x1_ref, y2_ref, x2_ref):
        ref[...] = jnp.zeros((4, 8, 128), jnp.float32)

    alive0 = (idx < _PRE).astype(jnp.float32)

    def body(t, alive4):
        onehot = (idx_o == t).astype(jnp.float32)
        new_alive = []
        for b in range(4):
            y1, x1, y2, x2, area = boxes[b]
            al = alive4[b]
            i = jnp.min(jnp.where(al > 0.0, idx, _BIG))
            valid = i < _BIG
            sel = idx == i

            def ext(v):
                return jnp.max(jnp.where(sel, v, -1e30))

            by1, bx1, by2, bx2, barea = (ext(v) for v in boxes[b])
            yy1 = jnp.maximum(by1, y1)
            xx1 = jnp.maximum(bx1, x1)
            yy2 = jnp.minimum(by2, y2)
            xx2 = jnp.minimum(bx2, x2)
            inter = jnp.maximum(yy2 - yy1, 0.0) * jnp.maximum(xx2 - xx1, 0.0)
            union = barea + area - inter
            iou = inter / jnp.maximum(union, 1e-8)
            al = jnp.where((iou > _THR) | sel, 0.0, al)
            new_alive.append(al)

            y1_ref[b] += onehot * jnp.where(valid, by1, 0.0)
            x1_ref[b] += onehot * jnp.where(valid, bx1, 0.0)
            y2_ref[b] += onehot * jnp.where(valid, by2, 0.0)
            x2_ref[b] += onehot * jnp.where(valid, bx2, 0.0)
        return tuple(new_alive)

    jax.lax.fori_loop(0, _NPROP, body, (alive0,) * 4)


def kernel(rpn_class, rpn_bbox, anchors):
    scores = rpn_class[:, :, 1]
    _, ix = jax.lax.top_k(scores, _PRE)
    a_g = jnp.take_along_axis(anchors, ix[..., None], axis=1)
    d_g = jnp.take_along_axis(rpn_bbox, ix[..., None], axis=1)

    def prep(x):
        x = jnp.pad(x, ((0, 0), (0, _PAD - _PRE), (0, 0)))
        return x.transpose(0, 2, 1).reshape(4, 4, _ROWS, 128)

    outs = pl.pallas_call(
        _nms_kernel,
        out_shape=[jax.ShapeDtypeStruct((4, 8, 128), jnp.float32)] * 4,
    )(prep(a_g), prep(d_g))
    planes = [o.reshape(4, 1024) for o in outs]
    return jnp.stack(planes, axis=-1)[:, :_NPROP, :]


# speedup vs baseline: 11.0459x; 1.0474x over previous
import functools

import jax
import jax.numpy as jnp
import numpy as np
from jax import lax
from jax.experimental import pallas as pl
from jax.experimental.pallas import tpu as pltpu
from jax.experimental.pallas import tpu_sc as plsc

_PRE = 6000
_NPROP = 1000
_THR = 0.7
_PAD = 6144
_ROWS = 48
_BIG = 1 << 30


def _nms_kernel(a_ref, d_ref, y1_ref, x1_ref, y2_ref, x2_ref):
    row = jax.lax.broadcasted_iota(jnp.int32, (_ROWS, 128), 0)
    col = jax.lax.broadcasted_iota(jnp.int32, (_ROWS, 128), 1)
    idx = row * 128 + col
    row_o = jax.lax.broadcasted_iota(jnp.int32, (8, 128), 0)
    col_o = jax.lax.broadcasted_iota(jnp.int32, (8, 128), 1)
    idx_o = row_o * 128 + col_o

    boxes = []
    for b in range(4):
        ay1, ax1, ay2, ax2 = (a_ref[b, c] for c in range(4))
        dy = d_ref[b, 0] * 0.1
        dx = d_ref[b, 1] * 0.1
        dh = d_ref[b, 2] * 0.2
        dw = d_ref[b, 3] * 0.2
        h = ay2 - ay1
        w = ax2 - ax1
        cy = ay1 + 0.5 * h + dy * h
        cx = ax1 + 0.5 * w + dx * w
        h = h * jnp.exp(dh)
        w = w * jnp.exp(dw)
        y1 = cy - 0.5 * h
        x1 = cx - 0.5 * w
        y2 = y1 + h
        x2 = x1 + w
        y1 = jnp.clip(y1, 0.0, 1.0)
        x1 = jnp.clip(x1, 0.0, 1.0)
        y2 = jnp.clip(y2, 0.0, 1.0)
        x2 = jnp.clip(x2, 0.0, 1.0)
        area = jnp.maximum(y2 - y1, 0.0) * jnp.maximum(x2 - x1, 0.0)
        boxes.append((y1, x1, y2, x2, area))

    for ref in (y1_ref, x1_ref, y2_ref, x2_ref):
        ref[...] = jnp.zeros((4, 8, 128), jnp.float32)

    alive0 = (idx < _PRE).astype(jnp.float32)

    def body(t, alive4):
        onehot = (idx_o == t).astype(jnp.float32)
        new_alive = []
        for b in range(4):
            y1, x1, y2, x2, area = boxes[b]
            al = alive4[b]
            i = jnp.min(jnp.where(al > 0.0, idx, _BIG))
            valid = i < _BIG
            sel = idx == i

            def ext(v):
                return jnp.max(jnp.where(sel, v, -1e30))

            by1, bx1, by2, bx2, barea = (ext(v) for v in boxes[b])
            yy1 = jnp.maximum(by1, y1)
            xx1 = jnp.maximum(bx1, x1)
            yy2 = jnp.minimum(by2, y2)
            xx2 = jnp.minimum(bx2, x2)
            inter = jnp.maximum(yy2 - yy1, 0.0) * jnp.maximum(xx2 - xx1, 0.0)
            union = barea + area - inter
            iou = inter / jnp.maximum(union, 1e-8)
            al = jnp.where((iou > _THR) | sel, 0.0, al)
            new_alive.append(al)

            y1_ref[b] += onehot * jnp.where(valid, by1, 0.0)
            x1_ref[b] += onehot * jnp.where(valid, bx1, 0.0)
            y2_ref[b] += onehot * jnp.where(valid, by2, 0.0)
            x2_ref[b] += onehot * jnp.where(valid, bx2, 0.0)
        return tuple(new_alive)

    jax.lax.fori_loop(0, _NPROP, body, (alive0,) * 4)


_NCHUNK = _PAD // 128


@functools.partial(
    pl.kernel,
    out_type=jax.ShapeDtypeStruct((32, _NCHUNK, 128), jnp.float32),
    mesh=plsc.VectorSubcoreMesh(core_axis_name="c", subcore_axis_name="s"),
    scratch_types=[
        pltpu.VMEM((_NCHUNK, 128), jnp.int32),
        pltpu.VMEM((_NCHUNK, 128), jnp.float32),
        pltpu.SemaphoreType.DMA,
    ],
)
def _sc_gather(flat_hbm, off_hbm, out_hbm, idx_v, rows_v, sem):
    wid = lax.axis_index("s") * 2 + lax.axis_index("c")
    pltpu.sync_copy(off_hbm.at[wid], idx_v)
    copies = [
        pltpu.make_async_copy(flat_hbm.at[idx_v.at[c]], rows_v.at[c], sem)
        for c in range(_NCHUNK)
    ]
    for cp in copies:
        cp.start()
    for cp in copies:
        cp.wait()
    pltpu.sync_copy(rows_v, out_hbm.at[wid])


def kernel(rpn_class, rpn_bbox, anchors):
    scores = rpn_class[:, :, 1]
    _, ix = jax.lax.top_k(scores, _PRE)
    ix = jnp.pad(ix, ((0, 0), (0, _PAD - _PRE)))

    planes = jnp.concatenate(
        [anchors.transpose(0, 2, 1), rpn_bbox.transpose(0, 2, 1)], axis=1
    )
    flat = planes.reshape(-1)
    base = (jnp.arange(4)[:, None] * 8 + jnp.arange(8)[None, :]) * 20000
    offs = (ix[:, None, :] + base[:, :, None]).astype(jnp.int32)
    offs = offs.reshape(32, _NCHUNK, 128)

    gathered = _sc_gather(flat, offs)
    g = gathered.reshape(4, 8, _ROWS, 128)

    outs = pl.pallas_call(
        _nms_kernel,
        out_shape=[jax.ShapeDtypeStruct((4, 8, 128), jnp.float32)] * 4,
    )(g[:, :4], g[:, 4:])
    planes = [o.reshape(4, 1024) for o in outs]
    return jnp.stack(planes, axis=-1)[:, :_NPROP, :]


# speedup vs baseline: 11.8532x; 1.0731x over previous
import functools

import jax
import jax.numpy as jnp
import numpy as np
from jax import lax
from jax.experimental import pallas as pl
from jax.experimental.pallas import tpu as pltpu
from jax.experimental.pallas import tpu_sc as plsc

_PRE = 6000
_NPROP = 1000
_THR = 0.7
_PAD = 6144
_ROWS = 48
_BIG = 1 << 30


def _nms_kernel(a_ref, d_ref, y1_ref, x1_ref, y2_ref, x2_ref):
    row = jax.lax.broadcasted_iota(jnp.int32, (_ROWS, 128), 0)
    col = jax.lax.broadcasted_iota(jnp.int32, (_ROWS, 128), 1)
    idx = row * 128 + col
    row_o = jax.lax.broadcasted_iota(jnp.int32, (8, 128), 0)
    col_o = jax.lax.broadcasted_iota(jnp.int32, (8, 128), 1)
    idx_o = row_o * 128 + col_o

    boxes = []
    for b in range(4):
        ay1, ax1, ay2, ax2 = (a_ref[b, c] for c in range(4))
        dy = d_ref[b, 0] * 0.1
        dx = d_ref[b, 1] * 0.1
        dh = d_ref[b, 2] * 0.2
        dw = d_ref[b, 3] * 0.2
        h = ay2 - ay1
        w = ax2 - ax1
        cy = ay1 + 0.5 * h + dy * h
        cx = ax1 + 0.5 * w + dx * w
        h = h * jnp.exp(dh)
        w = w * jnp.exp(dw)
        y1 = cy - 0.5 * h
        x1 = cx - 0.5 * w
        y2 = y1 + h
        x2 = x1 + w
        y1 = jnp.clip(y1, 0.0, 1.0)
        x1 = jnp.clip(x1, 0.0, 1.0)
        y2 = jnp.clip(y2, 0.0, 1.0)
        x2 = jnp.clip(x2, 0.0, 1.0)
        area = jnp.maximum(y2 - y1, 0.0) * jnp.maximum(x2 - x1, 0.0)
        boxes.append((y1, x1, y2, x2, area))

    for ref in (y1_ref, x1_ref, y2_ref, x2_ref):
        ref[...] = jnp.zeros((4, 8, 128), jnp.float32)

    alive0 = (idx < _PRE).astype(jnp.float32)

    def body(t, alive4):
        onehot = (idx_o == t).astype(jnp.float32)
        new_alive = []
        for b in range(4):
            y1, x1, y2, x2, area = boxes[b]
            al = alive4[b]
            i = jnp.min(jnp.where(al > 0.0, idx, _BIG))
            valid = i < _BIG
            sel = idx == i

            def ext(v):
                return jnp.max(jnp.where(sel, v, -1e30))

            by1, bx1, by2, bx2, barea = (ext(v) for v in boxes[b])
            yy1 = jnp.maximum(by1, y1)
            xx1 = jnp.maximum(bx1, x1)
            yy2 = jnp.minimum(by2, y2)
            xx2 = jnp.minimum(bx2, x2)
            inter = jnp.maximum(yy2 - yy1, 0.0) * jnp.maximum(xx2 - xx1, 0.0)
            union = barea + area - inter
            iou = inter / jnp.maximum(union, 1e-8)
            al = jnp.where((iou > _THR) | sel, 0.0, al)
            new_alive.append(al)

            y1_ref[b] += onehot * jnp.where(valid, by1, 0.0)
            x1_ref[b] += onehot * jnp.where(valid, bx1, 0.0)
            y2_ref[b] += onehot * jnp.where(valid, by2, 0.0)
            x2_ref[b] += onehot * jnp.where(valid, bx2, 0.0)
        return tuple(new_alive)

    jax.lax.fori_loop(0, _NPROP, body, (alive0,) * 4)


_SORTN = 32768
_SROWS = _SORTN // 128


def _sort_kernel(key_ref, idx_ref):
    key = key_ref[0]
    row = lax.broadcasted_iota(jnp.int32, (_SROWS, 128), 0)
    col = lax.broadcasted_iota(jnp.int32, (_SROWS, 128), 1)
    pos = row * 128 + col
    idx = pos
    k = 2
    while k <= _SORTN:
        j = k // 2
        while j >= 1:
            bitj = (pos & j) != 0
            hold_early = ((pos & k) == 0) == (~bitj)
            ax, s = (0, j // 128) if j >= 128 else (1, j)
            pk = jnp.where(bitj, jnp.roll(key, s, axis=ax), jnp.roll(key, -s, axis=ax))
            pi = jnp.where(bitj, jnp.roll(idx, s, axis=ax), jnp.roll(idx, -s, axis=ax))
            before = (pk > key) | ((pk == key) & (pi < idx))
            take = before == hold_early
            key = jnp.where(take, pk, key)
            idx = jnp.where(take, pi, idx)
            j //= 2
        k *= 2
    idx_ref[0] = idx


def _topk_indices(scores):
    pad = jnp.full((4, _SORTN - scores.shape[1]), -jnp.inf, jnp.float32)
    keys = jnp.concatenate([scores, pad], axis=1).reshape(4, _SROWS, 128)
    sorted_idx = pl.pallas_call(
        _sort_kernel,
        grid=(4,),
        in_specs=[pl.BlockSpec((1, _SROWS, 128), lambda b: (b, 0, 0))],
        out_specs=pl.BlockSpec((1, _SROWS, 128), lambda b: (b, 0, 0)),
        out_shape=jax.ShapeDtypeStruct((4, _SROWS, 128), jnp.int32),
    )(keys)
    return sorted_idx.reshape(4, _SORTN)[:, :_PAD]


_NCHUNK = _PAD // 128


def _sc_gather_body(flat_hbm, off_hbm, out_hbm, idx_v, rows_v, sem):
    wid = lax.axis_index("s") * 2 + lax.axis_index("c")
    pltpu.sync_copy(off_hbm.at[wid], idx_v)
    copies = [
        pltpu.make_async_copy(flat_hbm.at[idx_v.at[c]], rows_v.at[c], sem)
        for c in range(_NCHUNK)
    ]
    for cp in copies:
        cp.start()
    for cp in copies:
        cp.wait()
    pltpu.sync_copy(rows_v, out_hbm.at[wid])


@functools.cache
def _sc_gather():
    return pl.kernel(
        _sc_gather_body,
        out_type=jax.ShapeDtypeStruct((32, _NCHUNK, 128), jnp.float32),
        mesh=plsc.VectorSubcoreMesh(core_axis_name="c", subcore_axis_name="s"),
        scratch_types=[
            pltpu.VMEM((_NCHUNK, 128), jnp.int32),
            pltpu.VMEM((_NCHUNK, 128), jnp.float32),
            pltpu.SemaphoreType.DMA,
        ],
    )


def kernel(rpn_class, rpn_bbox, anchors):
    scores = rpn_class[:, :, 1]
    ix = _topk_indices(scores)

    planes = jnp.concatenate(
        [anchors.transpose(0, 2, 1), rpn_bbox.transpose(0, 2, 1)], axis=1
    )
    flat = planes.reshape(-1)
    base = (jnp.arange(4)[:, None] * 8 + jnp.arange(8)[None, :]) * 20000
    offs = (ix[:, None, :] + base[:, :, None]).astype(jnp.int32)
    offs = offs.reshape(32, _NCHUNK, 128)

    gathered = _sc_gather()(flat, offs)
    g = gathered.reshape(4, 8, _ROWS, 128)

    outs = pl.pallas_call(
        _nms_kernel,
        out_shape=[jax.ShapeDtypeStruct((4, 8, 128), jnp.float32)] * 4,
    )(g[:, :4], g[:, 4:])
    planes = [o.reshape(4, 1024) for o in outs]
    return jnp.stack(planes, axis=-1)[:, :_NPROP, :]
